# baseline (speedup 1.0000x reference)
import functools

import jax
import jax.numpy as jnp
from jax import lax
from jax.experimental import pallas as pl
from jax.experimental.pallas import tpu as pltpu

_CH = 200
_NBUF = 4


def _attention_mix(acc, hi, hmlp, avl_ref, avh_ref, avm_ref, att_ref, f):
    out_low = jnp.maximum(acc[:, :f], 0.0)
    out_high = jnp.maximum(hi[:, f:] - acc[:, f:], 0.0)
    out_mlp = hmlp
    l0 = jnp.dot(out_low, avl_ref[...], preferred_element_type=jnp.float32)
    l1 = jnp.dot(out_high, avh_ref[...], preferred_element_type=jnp.float32)
    l2 = jnp.dot(out_mlp, avm_ref[...], preferred_element_type=jnp.float32)
    g0 = jax.nn.sigmoid(l0)
    g1 = jax.nn.sigmoid(l1)
    g2 = jax.nn.sigmoid(l2)
    third = 1.0 / 3.0
    m0 = (g0 * att_ref[0, 0] + g1 * att_ref[1, 0] + g2 * att_ref[2, 0]) * third
    m1 = (g0 * att_ref[0, 1] + g1 * att_ref[1, 1] + g2 * att_ref[2, 1]) * third
    m2 = (g0 * att_ref[0, 2] + g1 * att_ref[1, 2] + g2 * att_ref[2, 2]) * third
    mx = jnp.maximum(jnp.maximum(m0, m1), m2)
    e0 = jnp.exp(m0 - mx)
    e1 = jnp.exp(m1 - mx)
    e2 = jnp.exp(m2 - mx)
    inv = 3.0 / (e0 + e1 + e2)
    return (e0 * inv) * out_low + (e1 * inv) * out_high + (e2 * inv) * out_mlp


def _body(adj_ref, x_ref,
          wl0_ref, wh0_ref, wm0_ref, avl0_ref, avh0_ref, avm0_ref,
          wl1_ref, wh1_ref, wm1_ref, avl1_ref, avh1_ref, avm1_ref,
          att0_ref, att1_ref,
          out_ref,
          ring, hcat0_s, aux_s, sems,
          *, n, f0, f1):
    c1 = f0
    c2 = f0 + 2 * f1
    pc = n // _CH
    total = 2 * pc

    def start_fetch(c):
        r = lax.rem(c, pc)
        b = lax.rem(c, _NBUF)
        pltpu.make_async_copy(
            adj_ref.at[pl.ds(r * _CH, _CH), :], ring.at[b], sems.at[b]).start()

    for c in range(_NBUF - 1):
        start_fetch(c)

    xb = x_ref[...]
    hcat0_s[:, :f0] = jnp.dot(xb, wl0_ref[...], preferred_element_type=jnp.float32)
    hcat0_s[:, f0:] = jnp.dot(xb, wh0_ref[...], preferred_element_type=jnp.float32)
    aux_s[:, :f0] = jnp.maximum(
        jnp.dot(xb, wm0_ref[...], preferred_element_type=jnp.float32), 0.0)

    def step(c, _):
        b = lax.rem(c, _NBUF)
        pltpu.make_async_copy(
            adj_ref.at[pl.ds(lax.rem(c, pc) * _CH, _CH), :],
            ring.at[b], sems.at[b]).wait()

        @pl.when(c + (_NBUF - 1) < total)
        def _next():
            start_fetch(c + (_NBUF - 1))

        @pl.when(c < pc)
        def _layer1():
            adj = ring[b]
            acc = jnp.dot(adj, hcat0_s[...], preferred_element_type=jnp.float32)
            hi = hcat0_s[pl.ds(c * _CH, _CH), :]
            hmlp = aux_s[pl.ds(c * _CH, _CH), :f0]
            fea = jnp.maximum(
                _attention_mix(acc, hi, hmlp, avl0_ref, avh0_ref, avm0_ref,
                               att0_ref, f0), 0.0)
            aux_s[pl.ds(c * _CH, _CH), c1:c1 + f1] = jnp.dot(
                fea, wl1_ref[...], preferred_element_type=jnp.float32)
            aux_s[pl.ds(c * _CH, _CH), c1 + f1:c2] = jnp.dot(
                fea, wh1_ref[...], preferred_element_type=jnp.float32)
            aux_s[pl.ds(c * _CH, _CH), c2:c2 + f1] = jnp.maximum(
                jnp.dot(fea, wm1_ref[...], preferred_element_type=jnp.float32),
                0.0)

        @pl.when(c >= pc)
        def _layer2():
            j = c - pc
            adj = ring[b]
            acc = jnp.dot(adj, aux_s[:, c1:c2],
                          preferred_element_type=jnp.float32)
            hi = aux_s[pl.ds(j * _CH, _CH), c1:c2]
            hmlp = aux_s[pl.ds(j * _CH, _CH), c2:c2 + f1]
            out_ref[pl.ds(j * _CH, _CH), :] = _attention_mix(
                acc, hi, hmlp, avl1_ref, avh1_ref, avm1_ref, att1_ref, f1)

        return ()

    lax.fori_loop(0, total, step, (), unroll=False)


def kernel(input, adj_low, adj_high, adj_low_unnormalized,
           w_low0, w_high0, w_mlp0, av_low0, av_high0, av_mlp0, att_vec0,
           w_low1, w_high1, w_mlp1, av_low1, av_high1, av_mlp1, att_vec1):
    n = adj_low.shape[0]
    d = input.shape[1]
    f0 = w_low0.shape[1]
    f1 = w_low1.shape[1]

    body = functools.partial(_body, n=n, f0=f0, f1=f1)
    vspec = pl.BlockSpec(memory_space=pltpu.MemorySpace.VMEM)
    return pl.pallas_call(
        body,
        in_specs=[
            pl.BlockSpec(memory_space=pl.ANY),
            vspec,
            vspec, vspec, vspec,
            vspec, vspec, vspec,
            vspec, vspec, vspec,
            vspec, vspec, vspec,
            pl.BlockSpec(memory_space=pltpu.MemorySpace.SMEM),
            pl.BlockSpec(memory_space=pltpu.MemorySpace.SMEM),
        ],
        out_specs=pl.BlockSpec(memory_space=pltpu.MemorySpace.VMEM),
        out_shape=jax.ShapeDtypeStruct((n, f1), jnp.float32),
        scratch_shapes=[
            pltpu.VMEM((_NBUF, _CH, n), jnp.float32),
            pltpu.VMEM((n, 2 * f0), jnp.float32),
            pltpu.VMEM((n, 2 * f0), jnp.float32),
            pltpu.SemaphoreType.DMA((_NBUF,)),
        ],
    )(adj_low, input,
      w_low0, w_high0, w_mlp0, av_low0, av_high0, av_mlp0,
      w_low1, w_high1, w_mlp1, av_low1, av_high1, av_mlp1,
      att_vec0, att_vec1)


# speedup vs baseline: 1.0067x; 1.0067x over previous
import functools

import jax
import jax.numpy as jnp
from jax import lax
from jax.experimental import pallas as pl
from jax.experimental.pallas import tpu as pltpu

_CH = 200
_NBUF = 4


def _attention_mix(acc, hi, hmlp, avl_ref, avh_ref, avm_ref, att_ref, f):
    out_low = jnp.maximum(acc[:, :f], 0.0)
    out_high = jnp.maximum(hi[:, f:] - acc[:, f:], 0.0)
    out_mlp = hmlp
    l0 = jnp.dot(out_low, avl_ref[...], preferred_element_type=jnp.float32)
    l1 = jnp.dot(out_high, avh_ref[...], preferred_element_type=jnp.float32)
    l2 = jnp.dot(out_mlp, avm_ref[...], preferred_element_type=jnp.float32)
    g0 = jax.nn.sigmoid(l0)
    g1 = jax.nn.sigmoid(l1)
    g2 = jax.nn.sigmoid(l2)
    third = 1.0 / 3.0
    m0 = (g0 * att_ref[0, 0] + g1 * att_ref[1, 0] + g2 * att_ref[2, 0]) * third
    m1 = (g0 * att_ref[0, 1] + g1 * att_ref[1, 1] + g2 * att_ref[2, 1]) * third
    m2 = (g0 * att_ref[0, 2] + g1 * att_ref[1, 2] + g2 * att_ref[2, 2]) * third
    mx = jnp.maximum(jnp.maximum(m0, m1), m2)
    e0 = jnp.exp(m0 - mx)
    e1 = jnp.exp(m1 - mx)
    e2 = jnp.exp(m2 - mx)
    inv = 3.0 / (e0 + e1 + e2)
    return (e0 * inv) * out_low + (e1 * inv) * out_high + (e2 * inv) * out_mlp


def _body(adj_ref, x_ref,
          wl0_ref, wh0_ref, wm0_ref, avl0_ref, avh0_ref, avm0_ref,
          wl1_ref, wh1_ref, wm1_ref, avl1_ref, avh1_ref, avm1_ref,
          att0_ref, att1_ref,
          out_ref,
          ring, hcat0_s, aux_s, sems,
          *, n, f0, f1):
    c1 = f0
    c2 = f0 + 2 * f1
    pc = n // _CH
    total = 2 * pc

    def start_fetch(c):
        r = lax.rem(c, pc)
        b = lax.rem(c, _NBUF)
        pltpu.make_async_copy(
            adj_ref.at[pl.ds(r * _CH, _CH), :], ring.at[b], sems.at[b]).start()

    for c in range(_NBUF - 1):
        start_fetch(c)

    xb = x_ref[...]
    hcat0_s[:, :f0] = jnp.dot(xb, wl0_ref[...], preferred_element_type=jnp.float32)
    hcat0_s[:, f0:] = jnp.dot(xb, wh0_ref[...], preferred_element_type=jnp.float32)
    aux_s[:, :f0] = jnp.maximum(
        jnp.dot(xb, wm0_ref[...], preferred_element_type=jnp.float32), 0.0)

    def step(c, _):
        b = lax.rem(c, _NBUF)
        pltpu.make_async_copy(
            adj_ref.at[pl.ds(lax.rem(c, pc) * _CH, _CH), :],
            ring.at[b], sems.at[b]).wait()

        @pl.when(c < pc)
        def _layer1():
            adj = ring[b]
            acc = jnp.dot(adj, hcat0_s[...], preferred_element_type=jnp.float32)
            hi = hcat0_s[pl.ds(c * _CH, _CH), :]
            hmlp = aux_s[pl.ds(c * _CH, _CH), :f0]
            fea = jnp.maximum(
                _attention_mix(acc, hi, hmlp, avl0_ref, avh0_ref, avm0_ref,
                               att0_ref, f0), 0.0)
            aux_s[pl.ds(c * _CH, _CH), c1:c1 + f1] = jnp.dot(
                fea, wl1_ref[...], preferred_element_type=jnp.float32)
            aux_s[pl.ds(c * _CH, _CH), c1 + f1:c2] = jnp.dot(
                fea, wh1_ref[...], preferred_element_type=jnp.float32)
            aux_s[pl.ds(c * _CH, _CH), c2:c2 + f1] = jnp.maximum(
                jnp.dot(fea, wm1_ref[...], preferred_element_type=jnp.float32),
                0.0)

        @pl.when(c >= pc)
        def _layer2():
            j = c - pc
            adj = ring[b]
            acc = jnp.dot(adj, aux_s[:, c1:c2],
                          preferred_element_type=jnp.float32)
            hi = aux_s[pl.ds(j * _CH, _CH), c1:c2]
            hmlp = aux_s[pl.ds(j * _CH, _CH), c2:c2 + f1]
            out_ref[pl.ds(j * _CH, _CH), :] = _attention_mix(
                acc, hi, hmlp, avl1_ref, avh1_ref, avm1_ref, att1_ref, f1)

        @pl.when(c + (_NBUF - 1) < total)
        def _next():
            start_fetch(c + (_NBUF - 1))

        return ()

    lax.fori_loop(0, total, step, (), unroll=False)


def kernel(input, adj_low, adj_high, adj_low_unnormalized,
           w_low0, w_high0, w_mlp0, av_low0, av_high0, av_mlp0, att_vec0,
           w_low1, w_high1, w_mlp1, av_low1, av_high1, av_mlp1, att_vec1):
    n = adj_low.shape[0]
    d = input.shape[1]
    f0 = w_low0.shape[1]
    f1 = w_low1.shape[1]

    body = functools.partial(_body, n=n, f0=f0, f1=f1)
    vspec = pl.BlockSpec(memory_space=pltpu.MemorySpace.VMEM)
    return pl.pallas_call(
        body,
        in_specs=[
            pl.BlockSpec(memory_space=pl.ANY),
            vspec,
            vspec, vspec, vspec,
            vspec, vspec, vspec,
            vspec, vspec, vspec,
            vspec, vspec, vspec,
            pl.BlockSpec(memory_space=pltpu.MemorySpace.SMEM),
            pl.BlockSpec(memory_space=pltpu.MemorySpace.SMEM),
        ],
        out_specs=pl.BlockSpec(memory_space=pltpu.MemorySpace.VMEM),
        out_shape=jax.ShapeDtypeStruct((n, f1), jnp.float32),
        scratch_shapes=[
            pltpu.VMEM((_NBUF, _CH, n), jnp.float32),
            pltpu.VMEM((n, 2 * f0), jnp.float32),
            pltpu.VMEM((n, 2 * f0), jnp.float32),
            pltpu.SemaphoreType.DMA((_NBUF,)),
        ],
    )(adj_low, input,
      w_low0, w_high0, w_mlp0, av_low0, av_high0, av_mlp0,
      w_low1, w_high1, w_mlp1, av_low1, av_high1, av_mlp1,
      att_vec0, att_vec1)
